# Initial kernel scaffold; baseline (speedup 1.0000x reference)
import jax
import jax.numpy as jnp
from jax.experimental import pallas as pl


def kernel(pose_code, codebook):
    raise NotImplementedError("write your pallas kernel here")


# speedup vs baseline: 4.8520x; 4.8520x over previous
import functools

import jax
import jax.numpy as jnp
import numpy as np
from jax import lax
from jax.experimental import pallas as pl
from jax.experimental.pallas import tpu as pltpu
from jax.experimental.pallas import tpu_sc as plsc

B, N, D = 16, 1024, 512
T = B * N
K = 8
CD = 64
V = 512

NC, NS = 2, 16
NW = NC * NS
TPW = T // NW
NG = TPW // 16

NPAIR = K * (K + 1) // 2

ROWS = 512


def _topk_body(x_ref, cb_ref, idx_ref, cn_ref):
    x = x_ref[...]
    col = lax.broadcasted_iota(jnp.int32, (ROWS, D), 1)
    colk = lax.broadcasted_iota(jnp.int32, (ROWS, K), 1)
    idx_mat = jnp.zeros((ROWS, K), jnp.int32)
    neg_inf = jnp.float32(-jnp.inf)
    for t in range(K):
        m = jnp.max(x, axis=1, keepdims=True)
        cand = jnp.where(x == m, col, D)
        sel = jnp.min(cand, axis=1, keepdims=True)
        idx_mat = jnp.where(colk == t, sel, idx_mat)
        x = jnp.where(col == sel, neg_inf, x)
    idx_ref[...] = idx_mat

    @pl.when(pl.program_id(0) == 0)
    def _():
        c = cb_ref[...]
        nrm = jnp.sqrt(jnp.sum(c * c, axis=1, keepdims=True))
        cn = c / jnp.maximum(nrm, jnp.float32(1e-12))
        cn_ref[...] = cn.astype(jnp.bfloat16).astype(jnp.float32)


_topk_call = pl.pallas_call(
    _topk_body,
    grid=(T // ROWS,),
    in_specs=[
        pl.BlockSpec((ROWS, D), lambda i: (i, 0)),
        pl.BlockSpec((V, CD), lambda i: (0, 0)),
    ],
    out_specs=[
        pl.BlockSpec((ROWS, K), lambda i: (i, 0)),
        pl.BlockSpec((V, CD), lambda i: (0, 0)),
    ],
    out_shape=[
        jax.ShapeDtypeStruct((T, K), jnp.int32),
        jax.ShapeDtypeStruct((V, CD), jnp.float32),
    ],
)


def _sc_pairs_body(cn_hbm, idx_hbm, out_hbm, cb_v, idx_v, out_v, sem):
    wid = lax.axis_index("s") * NC + lax.axis_index("c")
    pltpu.sync_copy(cn_hbm, cb_v)
    pltpu.sync_copy(idx_hbm.at[pl.ds(wid * (TPW * K), TPW * K)], idx_v)

    iota16 = lax.iota(jnp.int32, 16)

    def group_body(g, accs):
        gbase = iota16 * K + g * (16 * K)
        rbase = [plsc.load_gather(idx_v, [gbase + k]) * CD for k in range(K)]

        def d_body(dd, accs):
            gk = [plsc.load_gather(cb_v, [rb + dd]) for rb in rbase]
            new = []
            p = 0
            for k in range(K):
                for j in range(k, K):
                    new.append(accs[p] + gk[k] * gk[j])
                    p += 1
            return tuple(new)

        return lax.fori_loop(0, CD, d_body, accs)

    accs0 = tuple(jnp.zeros((16,), jnp.float32) for _ in range(NPAIR))
    accs = lax.fori_loop(0, NG, group_body, accs0)
    for p in range(NPAIR):
        out_v[p, :] = accs[p]
    pltpu.sync_copy(out_v, out_hbm.at[wid])


@functools.cache
def _sc_pairs_call():
    return pl.kernel(
        _sc_pairs_body,
        out_type=jax.ShapeDtypeStruct((NW, NPAIR, 16), jnp.float32),
        mesh=plsc.VectorSubcoreMesh(core_axis_name="c", subcore_axis_name="s"),
        compiler_params=pltpu.CompilerParams(needs_layout_passes=False),
        scratch_types=[
            pltpu.VMEM((V * CD,), jnp.float32),
            pltpu.VMEM((TPW * K,), jnp.int32),
            pltpu.VMEM((NPAIR, 16), jnp.float32),
            pltpu.SemaphoreType.DMA,
        ],
    )


def _loss_body(p_ref, w_ref, b_ref, o_ref):
    s = jnp.sum(jnp.sum(p_ref[...], axis=2), axis=0)
    mean = s * jnp.float32(1.0 / T)
    o_ref[...] = jnp.sum(jnp.abs(mean - b_ref[...][0]) * w_ref[...][0]).reshape(1, 1)


_loss_call = pl.pallas_call(
    _loss_body,
    out_shape=jax.ShapeDtypeStruct((1, 1), jnp.float32),
)

_W_NP = np.zeros((1, NPAIR), np.float32)
_B_NP = np.zeros((1, NPAIR), np.float32)
_p = 0
for _k in range(K):
    for _j in range(_k, K):
        _W_NP[0, _p] = 1.0 if _j == _k else 2.0
        _B_NP[0, _p] = 1.0 if _j == _k else 0.0
        _p += 1


def kernel(pose_code, codebook):
    pose_flat = pose_code.reshape(T, D)
    idx, cnorm = _topk_call(pose_flat, codebook)
    partials = _sc_pairs_call()(cnorm.reshape(-1), idx.reshape(-1))
    loss2d = _loss_call(partials, jnp.asarray(_W_NP), jnp.asarray(_B_NP))
    return loss2d[0, 0]


# speedup vs baseline: 6.5047x; 1.3406x over previous
import functools

import jax
import jax.numpy as jnp
import numpy as np
from jax import lax
from jax.experimental import pallas as pl
from jax.experimental.pallas import tpu as pltpu
from jax.experimental.pallas import tpu_sc as plsc

B, N, D = 16, 1024, 512
T = B * N
K = 8
CD = 64
V = 512

NC, NS = 2, 16
NW = NC * NS
TPW = T // NW
NG = TPW // 16

NPAIR = K * (K + 1) // 2

ROWS = 512


def _topk_body(x_ref, cb_ref, idx_ref, cn_ref):
    x = x_ref[...]
    col = lax.broadcasted_iota(jnp.int32, (ROWS, D), 1)
    colk = lax.broadcasted_iota(jnp.int32, (ROWS, K), 1)
    idx_mat = jnp.zeros((ROWS, K), jnp.int32)
    neg_inf = jnp.float32(-jnp.inf)
    for t in range(K):
        m = jnp.max(x, axis=1, keepdims=True)
        cand = jnp.where(x == m, col, D)
        sel = jnp.min(cand, axis=1, keepdims=True)
        idx_mat = jnp.where(colk == t, sel, idx_mat)
        x = jnp.where(col == sel, neg_inf, x)
    idx_ref[...] = idx_mat

    @pl.when(pl.program_id(0) == 0)
    def _():
        c = cb_ref[...]
        nrm = jnp.sqrt(jnp.sum(c * c, axis=1, keepdims=True))
        cn = c / jnp.maximum(nrm, jnp.float32(1e-12))
        cn_ref[...] = cn.astype(jnp.bfloat16)


_topk_call = pl.pallas_call(
    _topk_body,
    grid=(T // ROWS,),
    in_specs=[
        pl.BlockSpec((ROWS, D), lambda i: (i, 0)),
        pl.BlockSpec((V, CD), lambda i: (0, 0)),
    ],
    out_specs=[
        pl.BlockSpec((ROWS, K), lambda i: (i, 0)),
        pl.BlockSpec((V, CD), lambda i: (0, 0)),
    ],
    out_shape=[
        jax.ShapeDtypeStruct((T, K), jnp.int32),
        jax.ShapeDtypeStruct((V, CD), jnp.bfloat16),
    ],
)


NDP = CD // 2
DP_CHUNK = 8


def _sc_pairs_body(cbp_hbm, idx_hbm, out_hbm, cbp_v, idx_v, out_v, sem):
    wid = lax.axis_index("s") * NC + lax.axis_index("c")
    pltpu.sync_copy(cbp_hbm, cbp_v)
    pltpu.sync_copy(idx_hbm.at[pl.ds(wid * (TPW * K), TPW * K)], idx_v)

    iota16 = lax.iota(jnp.int32, 16)
    himask = jnp.full((16,), -65536, jnp.int32)

    def group_body(g, accs):
        gbase = iota16 * K + g * (16 * K)
        rbase = [plsc.load_gather(idx_v, [gbase + k]) * NDP for k in range(K)]

        def dp_body(c, accs):
            accs = list(accs)
            for u in range(DP_CHUNK):
                dp = c * DP_CHUNK + u
                ws = [plsc.load_gather(cbp_v, [rb + dp]) for rb in rbase]
                los = [plsc.bitcast(w << 16, jnp.float32) for w in ws]
                his = [plsc.bitcast(w & himask, jnp.float32) for w in ws]
                p = 0
                for k in range(K):
                    for j in range(k, K):
                        accs[p] = accs[p] + los[k] * los[j] + his[k] * his[j]
                        p += 1
            return tuple(accs)

        return lax.fori_loop(0, NDP // DP_CHUNK, dp_body, accs)

    accs0 = tuple(jnp.zeros((16,), jnp.float32) for _ in range(NPAIR))
    accs = lax.fori_loop(0, NG, group_body, accs0)
    for p in range(NPAIR):
        out_v[p, :] = accs[p]
    pltpu.sync_copy(out_v, out_hbm.at[wid])


@functools.cache
def _sc_pairs_call():
    return pl.kernel(
        _sc_pairs_body,
        out_type=jax.ShapeDtypeStruct((NW, NPAIR, 16), jnp.float32),
        mesh=plsc.VectorSubcoreMesh(core_axis_name="c", subcore_axis_name="s"),
        compiler_params=pltpu.CompilerParams(needs_layout_passes=False),
        scratch_types=[
            pltpu.VMEM((V * NDP,), jnp.int32),
            pltpu.VMEM((TPW * K,), jnp.int32),
            pltpu.VMEM((NPAIR, 16), jnp.float32),
            pltpu.SemaphoreType.DMA,
        ],
    )


def _loss_body(p_ref, w_ref, b_ref, o_ref):
    s = jnp.sum(jnp.sum(p_ref[...], axis=2), axis=0)
    mean = s * jnp.float32(1.0 / T)
    o_ref[...] = jnp.sum(jnp.abs(mean - b_ref[...][0]) * w_ref[...][0]).reshape(1, 1)


_loss_call = pl.pallas_call(
    _loss_body,
    out_shape=jax.ShapeDtypeStruct((1, 1), jnp.float32),
)

_W_NP = np.zeros((1, NPAIR), np.float32)
_B_NP = np.zeros((1, NPAIR), np.float32)
_p = 0
for _k in range(K):
    for _j in range(_k, K):
        _W_NP[0, _p] = 1.0 if _j == _k else 2.0
        _B_NP[0, _p] = 1.0 if _j == _k else 0.0
        _p += 1


def kernel(pose_code, codebook):
    pose_flat = pose_code.reshape(T, D)
    idx, cnorm_bf = _topk_call(pose_flat, codebook)
    packed = lax.bitcast_convert_type(
        cnorm_bf.reshape(V, NDP, 2), jnp.int32).reshape(-1)
    partials = _sc_pairs_call()(packed, idx.reshape(-1))
    loss2d = _loss_call(partials, jnp.asarray(_W_NP), jnp.asarray(_B_NP))
    return loss2d[0, 0]


# speedup vs baseline: 6.5166x; 1.0018x over previous
import functools

import jax
import jax.numpy as jnp
import numpy as np
from jax import lax
from jax.experimental import pallas as pl
from jax.experimental.pallas import tpu as pltpu
from jax.experimental.pallas import tpu_sc as plsc

B, N, D = 16, 1024, 512
T = B * N
K = 8
CD = 64
V = 512

NC, NS = 2, 16
NW = NC * NS
TPW = T // NW
NG = TPW // 16

NPAIR = K * (K + 1) // 2

ROWS = 512


def _topk_body(x_ref, cb_ref, idx_ref, cn_ref):
    x = x_ref[...]
    col = lax.broadcasted_iota(jnp.int32, (ROWS, D), 1)
    colk = lax.broadcasted_iota(jnp.int32, (ROWS, K), 1)
    idx_mat = jnp.zeros((ROWS, K), jnp.int32)
    neg_inf = jnp.float32(-jnp.inf)
    for t in range(K):
        m = jnp.max(x, axis=1, keepdims=True)
        cand = jnp.where(x == m, col, D)
        sel = jnp.min(cand, axis=1, keepdims=True)
        idx_mat = jnp.where(colk == t, sel, idx_mat)
        x = jnp.where(col == sel, neg_inf, x)
    idx_ref[...] = idx_mat

    @pl.when(pl.program_id(0) == 0)
    def _():
        c = cb_ref[...]
        nrm = jnp.sqrt(jnp.sum(c * c, axis=1, keepdims=True))
        cn = c / jnp.maximum(nrm, jnp.float32(1e-12))
        cn_ref[...] = cn.astype(jnp.bfloat16)


_topk_call = pl.pallas_call(
    _topk_body,
    grid=(T // ROWS,),
    in_specs=[
        pl.BlockSpec((ROWS, D), lambda i: (i, 0)),
        pl.BlockSpec((V, CD), lambda i: (0, 0)),
    ],
    out_specs=[
        pl.BlockSpec((ROWS, K), lambda i: (i, 0)),
        pl.BlockSpec((V, CD), lambda i: (0, 0)),
    ],
    out_shape=[
        jax.ShapeDtypeStruct((T, K), jnp.int32),
        jax.ShapeDtypeStruct((V, CD), jnp.bfloat16),
    ],
)


NDP = CD // 2
DP_CHUNK = 8


def _sc_pairs_body(cbp_hbm, idx_hbm, out_hbm, cbp_v, idx_v, out_v, sem):
    wid = lax.axis_index("s") * NC + lax.axis_index("c")
    pltpu.sync_copy(cbp_hbm, cbp_v)
    pltpu.sync_copy(idx_hbm.at[pl.ds(wid * (TPW * K), TPW * K)], idx_v)

    iota16 = lax.iota(jnp.int32, 16)
    himask = jnp.full((16,), -65536, jnp.int32)

    def group_body(g, accs):
        gbase = iota16 * K + g * (16 * K)
        rbase = [plsc.load_gather(idx_v, [gbase + k]) * NDP for k in range(K)]

        def dp_body(c, accs):
            accs = list(accs)
            for u in range(DP_CHUNK):
                dp = c * DP_CHUNK + u
                ws = [plsc.load_gather(cbp_v, [rb + dp]) for rb in rbase]
                los = [plsc.bitcast(w << 16, jnp.float32) for w in ws]
                p = 0
                for k in range(K):
                    for j in range(k, K):
                        accs[p] = accs[p] + los[k] * los[j]
                        p += 1
                his = [plsc.bitcast(w & himask, jnp.float32) for w in ws]
                p = 0
                for k in range(K):
                    for j in range(k, K):
                        accs[p] = accs[p] + his[k] * his[j]
                        p += 1
            return tuple(accs)

        return lax.fori_loop(0, NDP // DP_CHUNK, dp_body, accs)

    accs0 = tuple(jnp.zeros((16,), jnp.float32) for _ in range(NPAIR))
    accs = lax.fori_loop(0, NG, group_body, accs0)
    for p in range(NPAIR):
        out_v[p, :] = accs[p]
    pltpu.sync_copy(out_v, out_hbm.at[wid])


@functools.cache
def _sc_pairs_call():
    return pl.kernel(
        _sc_pairs_body,
        out_type=jax.ShapeDtypeStruct((NW, NPAIR, 16), jnp.float32),
        mesh=plsc.VectorSubcoreMesh(core_axis_name="c", subcore_axis_name="s"),
        compiler_params=pltpu.CompilerParams(needs_layout_passes=False),
        scratch_types=[
            pltpu.VMEM((V * NDP,), jnp.int32),
            pltpu.VMEM((TPW * K,), jnp.int32),
            pltpu.VMEM((NPAIR, 16), jnp.float32),
            pltpu.SemaphoreType.DMA,
        ],
    )


def _loss_body(p_ref, w_ref, b_ref, o_ref):
    s = jnp.sum(jnp.sum(p_ref[...], axis=2), axis=0)
    mean = s * jnp.float32(1.0 / T)
    o_ref[...] = jnp.sum(jnp.abs(mean - b_ref[...][0]) * w_ref[...][0]).reshape(1, 1)


_loss_call = pl.pallas_call(
    _loss_body,
    out_shape=jax.ShapeDtypeStruct((1, 1), jnp.float32),
)

_W_NP = np.zeros((1, NPAIR), np.float32)
_B_NP = np.zeros((1, NPAIR), np.float32)
_p = 0
for _k in range(K):
    for _j in range(_k, K):
        _W_NP[0, _p] = 1.0 if _j == _k else 2.0
        _B_NP[0, _p] = 1.0 if _j == _k else 0.0
        _p += 1


def kernel(pose_code, codebook):
    pose_flat = pose_code.reshape(T, D)
    idx, cnorm_bf = _topk_call(pose_flat, codebook)
    packed = lax.bitcast_convert_type(
        cnorm_bf.reshape(V, NDP, 2), jnp.int32).reshape(-1)
    partials = _sc_pairs_call()(packed, idx.reshape(-1))
    loss2d = _loss_call(partials, jnp.asarray(_W_NP), jnp.asarray(_B_NP))
    return loss2d[0, 0]


# speedup vs baseline: 7.5643x; 1.1608x over previous
import functools

import jax
import jax.numpy as jnp
import numpy as np
from jax import lax
from jax.experimental import pallas as pl
from jax.experimental.pallas import tpu as pltpu
from jax.experimental.pallas import tpu_sc as plsc

B, N, D = 16, 1024, 512
T = B * N
K = 8
CD = 64
V = 512

NC, NS = 2, 16
NW = NC * NS
TPW = T // NW
NG = TPW // 16

NPAIR = K * (K + 1) // 2

ROWS = 512


def _topk_body(x_ref, cb_ref, idx_ref, cn_ref):
    x = x_ref[...]
    col_f = lax.broadcasted_iota(jnp.int32, (ROWS, D), 1).astype(jnp.float32)
    neg_inf = jnp.float32(-jnp.inf)
    sels = []
    for t in range(K):
        m = jnp.max(x, axis=1, keepdims=True)
        cand = jnp.where(x == m, col_f, jnp.float32(1e9))
        sel_f = jnp.min(cand, axis=1, keepdims=True)
        sels.append(sel_f)
        x = jnp.where(col_f == sel_f, neg_inf, x)
    idx_ref[...] = jnp.concatenate(sels, axis=1).astype(jnp.int32)

    @pl.when(pl.program_id(0) == 0)
    def _():
        c = cb_ref[...]
        nrm = jnp.sqrt(jnp.sum(c * c, axis=1, keepdims=True))
        cn = c / jnp.maximum(nrm, jnp.float32(1e-12))
        cn_ref[...] = cn.astype(jnp.bfloat16)


_topk_call = pl.pallas_call(
    _topk_body,
    grid=(T // ROWS,),
    in_specs=[
        pl.BlockSpec((ROWS, D), lambda i: (i, 0)),
        pl.BlockSpec((V, CD), lambda i: (0, 0)),
    ],
    out_specs=[
        pl.BlockSpec((ROWS, K), lambda i: (i, 0)),
        pl.BlockSpec((V, CD), lambda i: (0, 0)),
    ],
    out_shape=[
        jax.ShapeDtypeStruct((T, K), jnp.int32),
        jax.ShapeDtypeStruct((V, CD), jnp.bfloat16),
    ],
)


NDP = CD // 2
DP_CHUNK = 8


def _sc_pairs_body(cbp_hbm, idx_hbm, out_hbm, cbp_v, idx_v, out_v, sem):
    wid = lax.axis_index("s") * NC + lax.axis_index("c")
    pltpu.sync_copy(cbp_hbm, cbp_v)
    pltpu.sync_copy(idx_hbm.at[pl.ds(wid * (TPW * K), TPW * K)], idx_v)

    iota16 = lax.iota(jnp.int32, 16)
    himask = jnp.full((16,), -65536, jnp.int32)

    def group_body(g, accs):
        gbase = iota16 * K + g * (16 * K)
        rbase = [plsc.load_gather(idx_v, [gbase + k]) * NDP for k in range(K)]

        def dp_body(c, accs):
            accs = list(accs)
            for u in range(DP_CHUNK):
                dp = c * DP_CHUNK + u
                ws = [plsc.load_gather(cbp_v, [rb + dp]) for rb in rbase]
                los = [plsc.bitcast(w << 16, jnp.float32) for w in ws]
                p = 0
                for k in range(K):
                    for j in range(k, K):
                        accs[p] = accs[p] + los[k] * los[j]
                        p += 1
                his = [plsc.bitcast(w & himask, jnp.float32) for w in ws]
                p = 0
                for k in range(K):
                    for j in range(k, K):
                        accs[p] = accs[p] + his[k] * his[j]
                        p += 1
            return tuple(accs)

        return lax.fori_loop(0, NDP // DP_CHUNK, dp_body, accs)

    accs0 = tuple(jnp.zeros((16,), jnp.float32) for _ in range(NPAIR))
    accs = lax.fori_loop(0, NG, group_body, accs0)
    for p in range(NPAIR):
        out_v[p, :] = accs[p]
    pltpu.sync_copy(out_v, out_hbm.at[wid])


@functools.cache
def _sc_pairs_call():
    return pl.kernel(
        _sc_pairs_body,
        out_type=jax.ShapeDtypeStruct((NW, NPAIR, 16), jnp.float32),
        mesh=plsc.VectorSubcoreMesh(core_axis_name="c", subcore_axis_name="s"),
        compiler_params=pltpu.CompilerParams(needs_layout_passes=False),
        scratch_types=[
            pltpu.VMEM((V * NDP,), jnp.int32),
            pltpu.VMEM((TPW * K,), jnp.int32),
            pltpu.VMEM((NPAIR, 16), jnp.float32),
            pltpu.SemaphoreType.DMA,
        ],
    )


def _loss_body(p_ref, w_ref, b_ref, o_ref):
    s = jnp.sum(jnp.sum(p_ref[...], axis=2), axis=0)
    mean = s * jnp.float32(1.0 / T)
    o_ref[...] = jnp.sum(jnp.abs(mean - b_ref[...][0]) * w_ref[...][0]).reshape(1, 1)


_loss_call = pl.pallas_call(
    _loss_body,
    out_shape=jax.ShapeDtypeStruct((1, 1), jnp.float32),
)

_W_NP = np.zeros((1, NPAIR), np.float32)
_B_NP = np.zeros((1, NPAIR), np.float32)
_p = 0
for _k in range(K):
    for _j in range(_k, K):
        _W_NP[0, _p] = 1.0 if _j == _k else 2.0
        _B_NP[0, _p] = 1.0 if _j == _k else 0.0
        _p += 1


def kernel(pose_code, codebook):
    pose_flat = pose_code.reshape(T, D)
    idx, cnorm_bf = _topk_call(pose_flat, codebook)
    packed = lax.bitcast_convert_type(
        cnorm_bf.reshape(V, NDP, 2), jnp.int32).reshape(-1)
    partials = _sc_pairs_call()(packed, idx.reshape(-1))
    loss2d = _loss_call(partials, jnp.asarray(_W_NP), jnp.asarray(_B_NP))
    return loss2d[0, 0]
